# baseline (speedup 1.0000x reference)
import functools

import jax
import jax.numpy as jnp
from jax import lax
from jax.experimental import pallas as pl
from jax.experimental.pallas import tpu as pltpu
from jax.experimental.pallas import tpu_sc as plsc

_LANES = 16
_NSLOTS = 4
_LEAD = 2


def _detile_body(bcols, x_ref, o_ref):
    h = bcols // 2
    d = x_ref.shape[0]
    x = x_ref[...]
    r = lax.broadcasted_iota(jnp.int32, (d, d), 0)
    c = lax.broadcasted_iota(jnp.int32, (d, d), 1)
    eye = jnp.where(r == c, 1.0, 0.0).astype(jnp.float32)
    dn = (((0,), (0,)), ((), ()))
    ta = lax.dot_general(x[:, :h], eye, dn,
                         preferred_element_type=jnp.float32)
    tb = lax.dot_general(x[:, h:], eye, dn,
                         preferred_element_type=jnp.float32)
    w = jnp.concatenate([ta, tb], axis=1)
    o_ref[...] = jnp.reshape(w, (bcols * d,))


def _detile_tc(table_t):
    d, v = table_t.shape
    bcols = 8192
    return pl.pallas_call(
        functools.partial(_detile_body, bcols),
        grid=(pl.cdiv(v, bcols),),
        in_specs=[pl.BlockSpec((d, bcols), lambda i: (0, i))],
        out_specs=pl.BlockSpec((bcols * d,), lambda i: (i,)),
        out_shape=jax.ShapeDtypeStruct((pl.cdiv(v, bcols) * bcols * d,),
                                       jnp.float32),
    )(table_t)


def _gather_sum_sc(ids, tgt, table):
    n_items, hist = ids.shape
    _, d = table.shape
    info = plsc.get_sparse_core_info()
    nc, ns = info.num_cores, info.num_subcores
    nw = nc * ns
    ipw = n_items // nw
    nvec = d // _LANES
    mesh = plsc.VectorSubcoreMesh(core_axis_name="c", subcore_axis_name="s")

    @functools.partial(
        pl.kernel,
        out_type=jax.ShapeDtypeStruct((n_items, d), jnp.float32),
        mesh=mesh,
        scratch_types=[
            pltpu.VMEM((ipw, hist), jnp.int32),
            pltpu.VMEM((ipw, hist), jnp.int32),
            pltpu.VMEM((ipw, hist), jnp.int32),
            pltpu.VMEM((_NSLOTS, hist, d), jnp.float32),
            pltpu.VMEM((ipw, d), jnp.float32),
            pltpu.VMEM_SHARED((ns * ipw, d), jnp.float32),
            pltpu.SemaphoreType.DMA,
            pltpu.SemaphoreType.DMA,
            pltpu.SemaphoreType.DMA,
            pltpu.SemaphoreType.DMA,
            pltpu.SemaphoreType.DMA,
            pltpu.SemaphoreType.DMA,
            pltpu.SemaphoreType.DMA,
            pltpu.SemaphoreType.DMA,
        ],
        compiler_params=pltpu.CompilerParams(use_tc_tiling_on_sc=False),
    )
    def body(ids_hbm, tgt_hbm, table_hbm, out_hbm, idx_v, idx2_v, tgt_v, buf,
             zeros_v, acc_s, *sems):
        sem_g, sem_s = sems[:_NSLOTS], sems[_NSLOTS:]
        sid = lax.axis_index("s")
        wid = sid * nc + lax.axis_index("c")
        pltpu.sync_copy(ids_hbm.at[pl.ds(wid * ipw, ipw)], idx_v)
        pltpu.sync_copy(tgt_hbm.at[sid], tgt_v)

        def perm(i, c):
            for o in (0, 16, 32, 34):
                raw = idx_v[i, pl.ds(o, _LANES)]
                u = lax.bitwise_and(raw, 8191)
                fr = (lax.bitwise_and(raw, -8192)
                      + lax.bitwise_and(u * 2, 8191)
                      + lax.shift_right_logical(u, 12))
                idx2_v[i, pl.ds(o, _LANES)] = fr
            return c

        lax.fori_loop(0, ipw, perm, 0)

        zeros = jnp.zeros((_LANES,), jnp.float32)

        def zbody(i, c):
            for j in range(nvec):
                zeros_v[i, pl.ds(_LANES * j, _LANES)] = zeros
            return c

        lax.fori_loop(0, ipw, zbody, 0)
        pltpu.sync_copy(zeros_v, acc_s.at[pl.ds(sid * ipw, ipw)])

        for c in range(_LEAD):
            pltpu.async_copy(table_hbm.at[idx2_v.at[c]], buf.at[c], sem_g[c])

        def steps(kk, carry):
            for b in range(_NSLOTS):
                k = kk * _NSLOTS + b
                pltpu.make_async_copy(
                    table_hbm.at[idx2_v.at[k]], buf.at[b], sem_g[b]).wait()
                pltpu.async_copy(
                    buf.at[b], acc_s.at[tgt_v.at[k]], sem_s[b], add=True)
                g = k + _LEAD
                bg = (b + _LEAD) % _NSLOTS

                @pl.when(g < ipw)
                def _():
                    @pl.when(g >= _NSLOTS)
                    def _():
                        pltpu.make_async_copy(
                            buf.at[bg], acc_s.at[tgt_v.at[k]], sem_s[bg]).wait()

                    pltpu.async_copy(
                        table_hbm.at[idx2_v.at[g]], buf.at[bg], sem_g[bg])
            return carry

        lax.fori_loop(0, ipw // _NSLOTS, steps, 0)

        for b in range(_NSLOTS):
            pltpu.make_async_copy(
                buf.at[b], acc_s.at[tgt_v.at[0]], sem_s[b]).wait()

        pltpu.sync_copy(acc_s.at[pl.ds(sid * ipw, ipw)],
                        out_hbm.at[pl.ds(wid * ipw, ipw)])

    return body(ids, tgt, table)


def _mm_body(scale, r_ref, t_ref, o_ref):
    o_ref[...] = jnp.dot(
        r_ref[...], t_ref[...], preferred_element_type=jnp.float32) * scale


def _matmul_tc(ratio, sums, scale):
    b, n = ratio.shape
    _, d = sums.shape
    bb = 256
    return pl.pallas_call(
        functools.partial(_mm_body, scale),
        grid=(b // bb,),
        in_specs=[
            pl.BlockSpec((bb, n), lambda i: (i, 0)),
            pl.BlockSpec((n, d), lambda i: (0, 0)),
        ],
        out_specs=pl.BlockSpec((bb, d), lambda i: (i, 0)),
        out_shape=jax.ShapeDtypeStruct((b, d), jnp.float32),
    )(ratio, sums)


def kernel(input_ids, input_ratio, embedding):
    n_items, hist = input_ids.shape
    vocab, d = embedding.shape
    info = plsc.get_sparse_core_info()
    ns = info.num_subcores
    ipw = n_items // (info.num_cores * ns)
    flat = _detile_tc(embedding.T)
    table_rm = flat.reshape(flat.shape[0] // d, d)
    tgt = jnp.broadcast_to(
        (jnp.arange(ns, dtype=jnp.int32)[:, None] * ipw
         + jnp.arange(ipw, dtype=jnp.int32)[None, :])[:, :, None],
        (ns, ipw, hist))
    sums = _gather_sum_sc(input_ids.astype(jnp.int32), tgt, table_rm)
    return _matmul_tc(input_ratio, sums, float(1.0 / hist))


# speedup vs baseline: 1.0329x; 1.0329x over previous
import functools

import jax
import jax.numpy as jnp
from jax import lax
from jax.experimental import pallas as pl
from jax.experimental.pallas import tpu as pltpu
from jax.experimental.pallas import tpu_sc as plsc

_LANES = 16
_NSLOTS = 4
_LEAD = 2


def _detile_body(bcols, x_ref, o_ref):
    h = bcols // 2
    d = x_ref.shape[0]
    x = x_ref[...]
    r = lax.broadcasted_iota(jnp.int32, (d, d), 0)
    c = lax.broadcasted_iota(jnp.int32, (d, d), 1)
    eye = jnp.where(r == c, 1.0, 0.0).astype(jnp.float32)
    dn = (((0,), (0,)), ((), ()))
    ta = lax.dot_general(x[:, :h], eye, dn,
                         preferred_element_type=jnp.float32)
    tb = lax.dot_general(x[:, h:], eye, dn,
                         preferred_element_type=jnp.float32)
    w = jnp.concatenate([ta, tb], axis=1)
    o_ref[...] = jnp.reshape(w, (bcols * d,))


def _detile_tc(table_t):
    d, v = table_t.shape
    bcols = 8192
    return pl.pallas_call(
        functools.partial(_detile_body, bcols),
        grid=(pl.cdiv(v, bcols),),
        in_specs=[pl.BlockSpec((d, bcols), lambda i: (0, i))],
        out_specs=pl.BlockSpec((bcols * d,), lambda i: (i,)),
        out_shape=jax.ShapeDtypeStruct((pl.cdiv(v, bcols) * bcols * d,),
                                       jnp.float32),
    )(table_t)


def _gather_sum_sc(ids, tgt, table):
    n_items, hist = ids.shape
    _, d = table.shape
    info = plsc.get_sparse_core_info()
    nc, ns = info.num_cores, info.num_subcores
    nw = nc * ns
    ipw = n_items // nw
    nvec = d // _LANES
    mesh = plsc.VectorSubcoreMesh(core_axis_name="c", subcore_axis_name="s")

    @functools.partial(
        pl.kernel,
        out_type=jax.ShapeDtypeStruct((n_items, d), jnp.float32),
        mesh=mesh,
        scratch_types=[
            pltpu.VMEM((ipw, hist), jnp.int32),
            pltpu.VMEM((ipw // 2, 2 * hist), jnp.int32),
            pltpu.VMEM((ipw // 2, 2 * hist), jnp.int32),
            pltpu.VMEM((_NSLOTS, 2 * hist, d), jnp.float32),
            pltpu.VMEM((ipw, d), jnp.float32),
            pltpu.VMEM_SHARED((ns * ipw, d), jnp.float32),
            pltpu.SemaphoreType.DMA,
            pltpu.SemaphoreType.DMA,
            pltpu.SemaphoreType.DMA,
            pltpu.SemaphoreType.DMA,
            pltpu.SemaphoreType.DMA,
            pltpu.SemaphoreType.DMA,
            pltpu.SemaphoreType.DMA,
            pltpu.SemaphoreType.DMA,
        ],
        compiler_params=pltpu.CompilerParams(use_tc_tiling_on_sc=False),
    )
    def body(ids_hbm, tgt_hbm, table_hbm, out_hbm, idx_v, idx2_v, tgt_v, buf,
             zeros_v, acc_s, *sems):
        sem_g, sem_s = sems[:_NSLOTS], sems[_NSLOTS:]
        sid = lax.axis_index("s")
        wid = sid * nc + lax.axis_index("c")
        pltpu.sync_copy(ids_hbm.at[pl.ds(wid * ipw, ipw)], idx_v)
        pltpu.sync_copy(tgt_hbm.at[sid], tgt_v)

        def perm(i, c):
            r2 = lax.shift_right_logical(i, 1)
            cb = lax.bitwise_and(i, 1) * hist
            for o in (0, 16, 32, 34):
                raw = idx_v[i, pl.ds(o, _LANES)]
                u = lax.bitwise_and(raw, 8191)
                fr = (lax.bitwise_and(raw, -8192)
                      + lax.bitwise_and(u * 2, 8191)
                      + lax.shift_right_logical(u, 12))
                idx2_v[r2, pl.ds(cb + o, _LANES)] = fr
            return c

        lax.fori_loop(0, ipw, perm, 0)

        zeros = jnp.zeros((_LANES,), jnp.float32)

        def zbody(i, c):
            for j in range(nvec):
                zeros_v[i, pl.ds(_LANES * j, _LANES)] = zeros
            return c

        lax.fori_loop(0, ipw, zbody, 0)
        pltpu.sync_copy(zeros_v, acc_s.at[pl.ds(sid * ipw, ipw)])

        for c in range(_LEAD):
            pltpu.async_copy(table_hbm.at[idx2_v.at[c]], buf.at[c], sem_g[c])

        def steps(kk, carry):
            for b in range(_NSLOTS):
                k = kk * _NSLOTS + b
                pltpu.make_async_copy(
                    table_hbm.at[idx2_v.at[k]], buf.at[b], sem_g[b]).wait()
                pltpu.async_copy(
                    buf.at[b], acc_s.at[tgt_v.at[k]], sem_s[b], add=True)
                g = k + _LEAD
                bg = (b + _LEAD) % _NSLOTS

                @pl.when(g < ipw // 2)
                def _():
                    @pl.when(g >= _NSLOTS)
                    def _():
                        pltpu.make_async_copy(
                            buf.at[bg], acc_s.at[tgt_v.at[k]], sem_s[bg]).wait()

                    pltpu.async_copy(
                        table_hbm.at[idx2_v.at[g]], buf.at[bg], sem_g[bg])
            return carry

        lax.fori_loop(0, ipw // 2 // _NSLOTS, steps, 0)

        for b in range(_NSLOTS):
            pltpu.make_async_copy(
                buf.at[b], acc_s.at[tgt_v.at[0]], sem_s[b]).wait()

        pltpu.sync_copy(acc_s.at[pl.ds(sid * ipw, ipw)],
                        out_hbm.at[pl.ds(wid * ipw, ipw)])

    return body(ids, tgt, table)


def _mm_body(scale, r_ref, t_ref, o_ref):
    o_ref[...] = jnp.dot(
        r_ref[...], t_ref[...], preferred_element_type=jnp.float32) * scale


def _matmul_tc(ratio, sums, scale):
    b, n = ratio.shape
    _, d = sums.shape
    bb = 256
    return pl.pallas_call(
        functools.partial(_mm_body, scale),
        grid=(b // bb,),
        in_specs=[
            pl.BlockSpec((bb, n), lambda i: (i, 0)),
            pl.BlockSpec((n, d), lambda i: (0, 0)),
        ],
        out_specs=pl.BlockSpec((bb, d), lambda i: (i, 0)),
        out_shape=jax.ShapeDtypeStruct((b, d), jnp.float32),
    )(ratio, sums)


def kernel(input_ids, input_ratio, embedding):
    n_items, hist = input_ids.shape
    vocab, d = embedding.shape
    info = plsc.get_sparse_core_info()
    ns = info.num_subcores
    ipw = n_items // (info.num_cores * ns)
    flat = _detile_tc(embedding.T)
    table_rm = flat.reshape(flat.shape[0] // d, d)
    tgt = jnp.broadcast_to(
        (jnp.arange(ns, dtype=jnp.int32)[:, None] * ipw
         + jnp.arange(ipw, dtype=jnp.int32)[None, :])[:, :, None],
        (ns, ipw, hist)).reshape(ns, ipw // 2, 2 * hist)
    sums = _gather_sum_sc(input_ids.astype(jnp.int32), tgt, table_rm)
    return _matmul_tc(input_ratio, sums, float(1.0 / hist))


# speedup vs baseline: 1.1446x; 1.1081x over previous
import functools

import jax
import jax.numpy as jnp
from jax import lax
from jax.experimental import pallas as pl
from jax.experimental.pallas import tpu as pltpu
from jax.experimental.pallas import tpu_sc as plsc

_LANES = 16
_NSLOTS = 4
_LEAD = 2


def _detile_body(bcols, x_ref, o_ref):
    h = bcols // 2
    d = x_ref.shape[0]
    x = x_ref[...]
    r = lax.broadcasted_iota(jnp.int32, (d, d), 0)
    c = lax.broadcasted_iota(jnp.int32, (d, d), 1)
    eye = jnp.where(r == c, 1.0, 0.0).astype(jnp.float32)
    dn = (((0,), (0,)), ((), ()))
    ta = lax.dot_general(x[:, :h], eye, dn,
                         preferred_element_type=jnp.float32)
    tb = lax.dot_general(x[:, h:], eye, dn,
                         preferred_element_type=jnp.float32)
    w = jnp.concatenate([ta, tb], axis=1)
    o_ref[...] = jnp.reshape(w, (bcols * d,))


def _detile_tc(table_t):
    d, v = table_t.shape
    bcols = 16384
    return pl.pallas_call(
        functools.partial(_detile_body, bcols),
        grid=(pl.cdiv(v, bcols),),
        in_specs=[pl.BlockSpec((d, bcols), lambda i: (0, i))],
        out_specs=pl.BlockSpec((bcols * d,), lambda i: (i,)),
        out_shape=jax.ShapeDtypeStruct((pl.cdiv(v, bcols) * bcols * d,),
                                       jnp.float32),
    )(table_t)


def _gather_sum_sc(ids, tgt, table):
    n_items, hist = ids.shape
    _, d = table.shape
    info = plsc.get_sparse_core_info()
    nc, ns = info.num_cores, info.num_subcores
    nw = nc * ns
    ipw = n_items // nw
    nvec = d // _LANES
    mesh = plsc.VectorSubcoreMesh(core_axis_name="c", subcore_axis_name="s")

    @functools.partial(
        pl.kernel,
        out_type=jax.ShapeDtypeStruct((n_items, d), jnp.float32),
        mesh=mesh,
        scratch_types=[
            pltpu.VMEM((ipw, hist), jnp.int32),
            pltpu.VMEM((ipw // 2, 2 * hist), jnp.int32),
            pltpu.VMEM((ipw // 2, 2 * hist), jnp.int32),
            pltpu.VMEM((_NSLOTS, 2 * hist, d), jnp.float32),
            pltpu.VMEM((ipw, d), jnp.float32),
            pltpu.VMEM_SHARED((ns * ipw, d), jnp.float32),
            pltpu.SemaphoreType.DMA,
            pltpu.SemaphoreType.DMA,
            pltpu.SemaphoreType.DMA,
            pltpu.SemaphoreType.DMA,
            pltpu.SemaphoreType.DMA,
            pltpu.SemaphoreType.DMA,
            pltpu.SemaphoreType.DMA,
            pltpu.SemaphoreType.DMA,
        ],
        compiler_params=pltpu.CompilerParams(use_tc_tiling_on_sc=False),
    )
    def body(ids_hbm, tgt_hbm, table_hbm, out_hbm, idx_v, idx2_v, tgt_v, buf,
             zeros_v, acc_s, *sems):
        sem_g, sem_s = sems[:_NSLOTS], sems[_NSLOTS:]
        sid = lax.axis_index("s")
        wid = sid * nc + lax.axis_index("c")
        pltpu.sync_copy(ids_hbm.at[pl.ds(wid * ipw, ipw)], idx_v)
        pltpu.sync_copy(tgt_hbm.at[sid], tgt_v)

        def perm(i, c):
            r2 = lax.shift_right_logical(i, 1)
            cb = lax.bitwise_and(i, 1) * hist
            for o in (0, 16, 32, 34):
                raw = idx_v[i, pl.ds(o, _LANES)]
                u = lax.bitwise_and(raw, 16383)
                fr = (lax.bitwise_and(raw, -16384)
                      + lax.bitwise_and(u * 2, 16383)
                      + lax.shift_right_logical(u, 13))
                idx2_v[r2, pl.ds(cb + o, _LANES)] = fr
            return c

        lax.fori_loop(0, ipw, perm, 0)

        zeros = jnp.zeros((_LANES,), jnp.float32)

        def zbody(i, c):
            for j in range(nvec):
                zeros_v[i, pl.ds(_LANES * j, _LANES)] = zeros
            return c

        lax.fori_loop(0, ipw, zbody, 0)
        pltpu.sync_copy(zeros_v, acc_s.at[pl.ds(sid * ipw, ipw)])

        for c in range(_LEAD):
            pltpu.async_copy(table_hbm.at[idx2_v.at[c]], buf.at[c], sem_g[c])

        def steps(kk, carry):
            for b in range(_NSLOTS):
                k = kk * _NSLOTS + b
                pltpu.make_async_copy(
                    table_hbm.at[idx2_v.at[k]], buf.at[b], sem_g[b]).wait()
                pltpu.async_copy(
                    buf.at[b], acc_s.at[tgt_v.at[k]], sem_s[b], add=True)
                g = k + _LEAD
                bg = (b + _LEAD) % _NSLOTS

                @pl.when(g < ipw // 2)
                def _():
                    @pl.when(g >= _NSLOTS)
                    def _():
                        pltpu.make_async_copy(
                            buf.at[bg], acc_s.at[tgt_v.at[k]], sem_s[bg]).wait()

                    pltpu.async_copy(
                        table_hbm.at[idx2_v.at[g]], buf.at[bg], sem_g[bg])
            return carry

        lax.fori_loop(0, ipw // 2 // _NSLOTS, steps, 0)

        for b in range(_NSLOTS):
            pltpu.make_async_copy(
                buf.at[b], acc_s.at[tgt_v.at[0]], sem_s[b]).wait()

        pltpu.sync_copy(acc_s.at[pl.ds(sid * ipw, ipw)],
                        out_hbm.at[pl.ds(wid * ipw, ipw)])

    return body(ids, tgt, table)


def _mm_body(scale, r_ref, t_ref, o_ref):
    o_ref[...] = jnp.dot(
        r_ref[...], t_ref[...], preferred_element_type=jnp.float32) * scale


def _matmul_tc(ratio, sums, scale):
    b, n = ratio.shape
    _, d = sums.shape
    bb = 256
    return pl.pallas_call(
        functools.partial(_mm_body, scale),
        grid=(b // bb,),
        in_specs=[
            pl.BlockSpec((bb, n), lambda i: (i, 0)),
            pl.BlockSpec((n, d), lambda i: (0, 0)),
        ],
        out_specs=pl.BlockSpec((bb, d), lambda i: (i, 0)),
        out_shape=jax.ShapeDtypeStruct((b, d), jnp.float32),
    )(ratio, sums)


def kernel(input_ids, input_ratio, embedding):
    n_items, hist = input_ids.shape
    vocab, d = embedding.shape
    info = plsc.get_sparse_core_info()
    ns = info.num_subcores
    ipw = n_items // (info.num_cores * ns)
    flat = _detile_tc(embedding.T)
    table_rm = flat.reshape(flat.shape[0] // d, d)
    tgt = jnp.broadcast_to(
        (jnp.arange(ns, dtype=jnp.int32)[:, None] * ipw
         + jnp.arange(ipw, dtype=jnp.int32)[None, :])[:, :, None],
        (ns, ipw, hist)).reshape(ns, ipw // 2, 2 * hist)
    sums = _gather_sum_sc(input_ids.astype(jnp.int32), tgt, table_rm)
    return _matmul_tc(input_ratio, sums, float(1.0 / hist))


# speedup vs baseline: 1.2015x; 1.0497x over previous
import functools

import jax
import jax.numpy as jnp
from jax import lax
from jax.experimental import pallas as pl
from jax.experimental.pallas import tpu as pltpu
from jax.experimental.pallas import tpu_sc as plsc

_LANES = 16
_NSLOTS = 4
_LEAD = 2


def _detile_body(bcols, x_ref, o_ref):
    h = bcols // 2
    d = x_ref.shape[0]
    x = x_ref[...]
    r = lax.broadcasted_iota(jnp.int32, (d, d), 0)
    c = lax.broadcasted_iota(jnp.int32, (d, d), 1)
    eye = jnp.where(r == c, 1.0, 0.0).astype(jnp.float32)
    dn = (((0,), (0,)), ((), ()))
    ta = lax.dot_general(x[:, :h], eye, dn,
                         preferred_element_type=jnp.float32)
    tb = lax.dot_general(x[:, h:], eye, dn,
                         preferred_element_type=jnp.float32)
    w = jnp.concatenate([ta, tb], axis=1)
    o_ref[...] = jnp.reshape(w, (bcols * d,))


def _detile_tc(table_t):
    d, v = table_t.shape
    bcols = 32768
    return pl.pallas_call(
        functools.partial(_detile_body, bcols),
        grid=(pl.cdiv(v, bcols),),
        in_specs=[pl.BlockSpec((d, bcols), lambda i: (0, i))],
        out_specs=pl.BlockSpec((bcols * d,), lambda i: (i,)),
        out_shape=jax.ShapeDtypeStruct((pl.cdiv(v, bcols) * bcols * d,),
                                       jnp.float32),
    )(table_t)


def _gather_sum_sc(ids, tgt, table):
    n_items, hist = ids.shape
    _, d = table.shape
    info = plsc.get_sparse_core_info()
    nc, ns = info.num_cores, info.num_subcores
    nw = nc * ns
    ipw = n_items // nw
    nvec = d // _LANES
    mesh = plsc.VectorSubcoreMesh(core_axis_name="c", subcore_axis_name="s")

    @functools.partial(
        pl.kernel,
        out_type=jax.ShapeDtypeStruct((n_items, d), jnp.float32),
        mesh=mesh,
        scratch_types=[
            pltpu.VMEM((ipw, hist), jnp.int32),
            pltpu.VMEM((ipw // 2, 2 * hist), jnp.int32),
            pltpu.VMEM((ipw // 2, 2 * hist), jnp.int32),
            pltpu.VMEM((_NSLOTS, 2 * hist, d), jnp.float32),
            pltpu.VMEM((ipw, d), jnp.float32),
            pltpu.VMEM_SHARED((ns * ipw, d), jnp.float32),
            pltpu.SemaphoreType.DMA,
            pltpu.SemaphoreType.DMA,
            pltpu.SemaphoreType.DMA,
            pltpu.SemaphoreType.DMA,
            pltpu.SemaphoreType.DMA,
            pltpu.SemaphoreType.DMA,
            pltpu.SemaphoreType.DMA,
            pltpu.SemaphoreType.DMA,
        ],
        compiler_params=pltpu.CompilerParams(use_tc_tiling_on_sc=False),
    )
    def body(ids_hbm, tgt_hbm, table_hbm, out_hbm, idx_v, idx2_v, tgt_v, buf,
             zeros_v, acc_s, *sems):
        sem_g, sem_s = sems[:_NSLOTS], sems[_NSLOTS:]
        sid = lax.axis_index("s")
        wid = sid * nc + lax.axis_index("c")
        pltpu.sync_copy(ids_hbm.at[pl.ds(wid * ipw, ipw)], idx_v)
        pltpu.sync_copy(tgt_hbm.at[sid], tgt_v)

        def perm(i, c):
            r2 = lax.shift_right_logical(i, 1)
            cb = lax.bitwise_and(i, 1) * hist
            for o in (0, 16, 32, 34):
                raw = idx_v[i, pl.ds(o, _LANES)]
                u = lax.bitwise_and(raw, 32767)
                fr = (lax.bitwise_and(raw, -32768)
                      + lax.bitwise_and(u * 2, 32767)
                      + lax.shift_right_logical(u, 14))
                idx2_v[r2, pl.ds(cb + o, _LANES)] = fr
            return c

        lax.fori_loop(0, ipw, perm, 0)

        zeros = jnp.zeros((_LANES,), jnp.float32)

        def zbody(i, c):
            for j in range(nvec):
                zeros_v[i, pl.ds(_LANES * j, _LANES)] = zeros
            return c

        lax.fori_loop(0, ipw, zbody, 0)
        pltpu.sync_copy(zeros_v, acc_s.at[pl.ds(sid * ipw, ipw)])

        for c in range(_LEAD):
            pltpu.async_copy(table_hbm.at[idx2_v.at[c]], buf.at[c], sem_g[c])

        def steps(kk, carry):
            for b in range(_NSLOTS):
                k = kk * _NSLOTS + b
                pltpu.make_async_copy(
                    table_hbm.at[idx2_v.at[k]], buf.at[b], sem_g[b]).wait()
                pltpu.async_copy(
                    buf.at[b], acc_s.at[tgt_v.at[k]], sem_s[b], add=True)
                g = k + _LEAD
                bg = (b + _LEAD) % _NSLOTS

                @pl.when(g < ipw // 2)
                def _():
                    @pl.when(g >= _NSLOTS)
                    def _():
                        pltpu.make_async_copy(
                            buf.at[bg], acc_s.at[tgt_v.at[k]], sem_s[bg]).wait()

                    pltpu.async_copy(
                        table_hbm.at[idx2_v.at[g]], buf.at[bg], sem_g[bg])
            return carry

        lax.fori_loop(0, ipw // 2 // _NSLOTS, steps, 0)

        for b in range(_NSLOTS):
            pltpu.make_async_copy(
                buf.at[b], acc_s.at[tgt_v.at[0]], sem_s[b]).wait()

        pltpu.sync_copy(acc_s.at[pl.ds(sid * ipw, ipw)],
                        out_hbm.at[pl.ds(wid * ipw, ipw)])

    return body(ids, tgt, table)


def _mm_body(scale, r_ref, t_ref, o_ref):
    o_ref[...] = jnp.dot(
        r_ref[...], t_ref[...], preferred_element_type=jnp.float32) * scale


def _matmul_tc(ratio, sums, scale):
    b, n = ratio.shape
    _, d = sums.shape
    bb = 256
    return pl.pallas_call(
        functools.partial(_mm_body, scale),
        grid=(b // bb,),
        in_specs=[
            pl.BlockSpec((bb, n), lambda i: (i, 0)),
            pl.BlockSpec((n, d), lambda i: (0, 0)),
        ],
        out_specs=pl.BlockSpec((bb, d), lambda i: (i, 0)),
        out_shape=jax.ShapeDtypeStruct((b, d), jnp.float32),
    )(ratio, sums)


def kernel(input_ids, input_ratio, embedding):
    n_items, hist = input_ids.shape
    vocab, d = embedding.shape
    info = plsc.get_sparse_core_info()
    ns = info.num_subcores
    ipw = n_items // (info.num_cores * ns)
    flat = _detile_tc(embedding.T)
    table_rm = flat.reshape(flat.shape[0] // d, d)
    tgt = jnp.broadcast_to(
        (jnp.arange(ns, dtype=jnp.int32)[:, None] * ipw
         + jnp.arange(ipw, dtype=jnp.int32)[None, :])[:, :, None],
        (ns, ipw, hist)).reshape(ns, ipw // 2, 2 * hist)
    sums = _gather_sum_sc(input_ids.astype(jnp.int32), tgt, table_rm)
    return _matmul_tc(input_ratio, sums, float(1.0 / hist))
